# baseline (speedup 1.0000x reference)
import functools

import jax
import jax.numpy as jnp
from jax.experimental import pallas as pl
from jax.experimental.pallas import tpu as pltpu


def _ohem_body(lp_hbm, lt_hbm, ct_ref, s_ref, n_ref,
               lp_buf, lt_buf, sem_lp, sem_lt, *, ta, chunks):
    i = pl.program_id(0)
    num = pl.num_programs(0)

    def copies(idx, slot):
        b = idx // chunks
        a0 = (idx % chunks) * ta
        src = (b, pl.ds(a0, ta), slice(None))
        return (
            pltpu.make_async_copy(lp_hbm.at[src], lp_buf.at[slot], sem_lp.at[slot]),
            pltpu.make_async_copy(lt_hbm.at[src], lt_buf.at[slot], sem_lt.at[slot]),
        )

    @pl.when(i == 0)
    def _prologue():
        for c in copies(0, 0):
            c.start()

    @pl.when(i + 1 < num)
    def _prefetch():
        for c in copies(i + 1, (i + 1) % 2):
            c.start()

    slot = i % 2
    for c in copies(i, slot):
        c.wait()

    diff = lp_buf[slot] - lt_buf[slot]
    diff_t = jnp.transpose(diff, (1, 0))

    absd = jnp.abs(diff_t)
    sl1 = jnp.where(absd < 1.0, 0.5 * diff_t * diff_t, absd - 0.5)

    b = i // chunks
    a0 = (i % chunks) * ta
    pos = (ct_ref[pl.ds(b, 1), pl.ds(a0, ta)] > 0).astype(jnp.float32)

    s_part = jnp.sum(sl1 * pos).reshape(1, 1)
    n_part = jnp.sum(pos).reshape(1, 1)

    @pl.when(i == 0)
    def _init():
        s_ref[...] = jnp.zeros((1, 1), jnp.float32)
        n_ref[...] = jnp.zeros((1, 1), jnp.float32)

    s_ref[...] += s_part
    n_ref[...] += n_part


def kernel(loc_preds, loc_targets, cls_preds, cls_targets):
    del cls_preds
    B, A, K = loc_preds.shape

    ta = 16384
    while A % ta:
        ta //= 2
    chunks = A // ta
    grid = B * chunks

    body = functools.partial(_ohem_body, ta=ta, chunks=chunks)

    s, n = pl.pallas_call(
        body,
        grid=(grid,),
        in_specs=[
            pl.BlockSpec(memory_space=pl.ANY),
            pl.BlockSpec(memory_space=pl.ANY),
            pl.BlockSpec((B, A), lambda i: (0, 0)),
        ],
        out_specs=[
            pl.BlockSpec((1, 1), lambda i: (0, 0)),
            pl.BlockSpec((1, 1), lambda i: (0, 0)),
        ],
        out_shape=[
            jax.ShapeDtypeStruct((1, 1), jnp.float32),
            jax.ShapeDtypeStruct((1, 1), jnp.float32),
        ],
        scratch_shapes=[
            pltpu.VMEM((2, ta, K), jnp.float32),
            pltpu.VMEM((2, ta, K), jnp.float32),
            pltpu.SemaphoreType.DMA((2,)),
            pltpu.SemaphoreType.DMA((2,)),
        ],
    )(loc_preds, loc_targets, cls_targets)

    loc_loss = s[0, 0]
    num_pos = n[0, 0]
    return 0.2 * loc_loss / num_pos


# speedup vs baseline: 2.0402x; 2.0402x over previous
import functools

import jax
import jax.numpy as jnp
from jax.experimental import pallas as pl


_TILE = 1024


def _ohem_body(lp_ref, lt_ref, ct_ref, s_ref, n_ref, *, k):
    i = pl.program_id(0)

    diff = lp_ref[...] - lt_ref[...]
    absd = jnp.abs(diff)
    sl1 = jnp.where(absd < 1.0, 0.5 * diff * diff, absd - 0.5)

    pos = (ct_ref[...] > 0).astype(jnp.float32)

    ej = jax.lax.broadcasted_iota(jnp.int32, (_TILE // k, _TILE), 0)
    el = jax.lax.broadcasted_iota(jnp.int32, (_TILE // k, _TILE), 1)
    e = (el // k == ej).astype(jnp.float32)

    chunk = sl1.shape[1]
    s_acc = jnp.zeros((), jnp.float32)
    for t in range(chunk // _TILE):
        pos_t = pos[:, t * (_TILE // k):(t + 1) * (_TILE // k)]
        mask_t = jax.lax.dot(pos_t, e, precision=jax.lax.Precision.HIGHEST)
        s_acc += jnp.sum(sl1[:, t * _TILE:(t + 1) * _TILE] * mask_t)

    s_part = s_acc.reshape(1, 1)
    n_part = jnp.sum(pos).reshape(1, 1)

    @pl.when(i == 0)
    def _init():
        s_ref[...] = jnp.zeros((1, 1), jnp.float32)
        n_ref[...] = jnp.zeros((1, 1), jnp.float32)

    s_ref[...] += s_part
    n_ref[...] += n_part


def kernel(loc_preds, loc_targets, cls_preds, cls_targets):
    del cls_preds
    B, A, K = loc_preds.shape
    F = A * K

    lp = loc_preds.reshape(B, F)
    lt = loc_targets.reshape(B, F)

    chunk = 65536
    while F % chunk:
        chunk //= 2
    cchunk = chunk // K
    grid = (B // 8) * (F // chunk)
    fchunks = F // chunk

    body = functools.partial(_ohem_body, k=K)

    s, n = pl.pallas_call(
        body,
        grid=(grid,),
        in_specs=[
            pl.BlockSpec((8, chunk), lambda i: (i // fchunks, i % fchunks)),
            pl.BlockSpec((8, chunk), lambda i: (i // fchunks, i % fchunks)),
            pl.BlockSpec((8, cchunk), lambda i: (i // fchunks, i % fchunks)),
        ],
        out_specs=[
            pl.BlockSpec((1, 1), lambda i: (0, 0)),
            pl.BlockSpec((1, 1), lambda i: (0, 0)),
        ],
        out_shape=[
            jax.ShapeDtypeStruct((1, 1), jnp.float32),
            jax.ShapeDtypeStruct((1, 1), jnp.float32),
        ],
    )(lp, lt, cls_targets)

    loc_loss = s[0, 0]
    num_pos = n[0, 0]
    return 0.2 * loc_loss / num_pos


# speedup vs baseline: 2.7924x; 1.3687x over previous
import functools

import jax
import jax.numpy as jnp
from jax.experimental import pallas as pl


def _ohem_body(lp_ref, lt_ref, ct_ref, s_ref, n_ref, *, k):
    i = pl.program_id(0)
    apr = 128 // k

    diff = lp_ref[0] - lt_ref[0]
    absd = jnp.abs(diff)
    sl1 = jnp.where(absd < 1.0, 0.5 * diff * diff, absd - 0.5)

    pos = (ct_ref[0] > 0).astype(jnp.float32)

    ej = jax.lax.broadcasted_iota(jnp.int32, (apr, 128), 0)
    el = jax.lax.broadcasted_iota(jnp.int32, (apr, 128), 1)
    e = (el // k == ej).astype(jnp.float32)

    maskexp = jax.lax.dot(pos, e, precision=jax.lax.Precision.HIGHEST)

    s_part = jnp.sum(sl1 * maskexp).reshape(1, 1)
    n_part = jnp.sum(pos).reshape(1, 1)

    @pl.when(i == 0)
    def _init():
        s_ref[...] = jnp.zeros((1, 1), jnp.float32)
        n_ref[...] = jnp.zeros((1, 1), jnp.float32)

    s_ref[...] += s_part
    n_ref[...] += n_part


def kernel(loc_preds, loc_targets, cls_preds, cls_targets):
    del cls_preds
    B, A, K = loc_preds.shape
    apr = 128 // K
    G = A // apr

    lp = loc_preds.reshape(B, G, 128)
    lt = loc_targets.reshape(B, G, 128)
    ct = cls_targets.reshape(B, G, apr)

    body = functools.partial(_ohem_body, k=K)

    s, n = pl.pallas_call(
        body,
        grid=(B,),
        in_specs=[
            pl.BlockSpec((1, G, 128), lambda i: (i, 0, 0)),
            pl.BlockSpec((1, G, 128), lambda i: (i, 0, 0)),
            pl.BlockSpec((1, G, apr), lambda i: (i, 0, 0)),
        ],
        out_specs=[
            pl.BlockSpec((1, 1), lambda i: (0, 0)),
            pl.BlockSpec((1, 1), lambda i: (0, 0)),
        ],
        out_shape=[
            jax.ShapeDtypeStruct((1, 1), jnp.float32),
            jax.ShapeDtypeStruct((1, 1), jnp.float32),
        ],
    )(lp, lt, ct)

    loc_loss = s[0, 0]
    num_pos = n[0, 0]
    return 0.2 * loc_loss / num_pos


# speedup vs baseline: 13.7152x; 4.9116x over previous
import functools

import jax
import jax.numpy as jnp
from jax.experimental import pallas as pl


def _ohem_body(lp_ref, lt_ref, ct_ref, s_ref, n_ref, *, ta, chunks):
    i = pl.program_id(0)
    b = i // chunks
    a0 = (i % chunks) * ta

    diff = lp_ref[0] - lt_ref[0]
    absd = jnp.abs(diff)
    sl1 = jnp.where(absd < 1.0, 0.5 * diff * diff, absd - 0.5)

    pos = (ct_ref[pl.ds(b, 1), pl.ds(a0, ta)] > 0).astype(jnp.float32)

    s_part = jnp.sum(sl1 * pos).reshape(1, 1)
    n_part = jnp.sum(pos).reshape(1, 1)

    @pl.when(i == 0)
    def _init():
        s_ref[...] = jnp.zeros((1, 1), jnp.float32)
        n_ref[...] = jnp.zeros((1, 1), jnp.float32)

    s_ref[...] += s_part
    n_ref[...] += n_part


def kernel(loc_preds, loc_targets, cls_preds, cls_targets):
    del cls_preds
    B, A, K = loc_preds.shape

    lp = jnp.transpose(loc_preds, (0, 2, 1))
    lt = jnp.transpose(loc_targets, (0, 2, 1))

    ta = 16384
    while A % ta:
        ta //= 2
    chunks = A // ta
    grid = B * chunks

    body = functools.partial(_ohem_body, ta=ta, chunks=chunks)

    s, n = pl.pallas_call(
        body,
        grid=(grid,),
        in_specs=[
            pl.BlockSpec((1, K, ta), lambda i: (i // chunks, 0, i % chunks)),
            pl.BlockSpec((1, K, ta), lambda i: (i // chunks, 0, i % chunks)),
            pl.BlockSpec((B, A), lambda i: (0, 0)),
        ],
        out_specs=[
            pl.BlockSpec((1, 1), lambda i: (0, 0)),
            pl.BlockSpec((1, 1), lambda i: (0, 0)),
        ],
        out_shape=[
            jax.ShapeDtypeStruct((1, 1), jnp.float32),
            jax.ShapeDtypeStruct((1, 1), jnp.float32),
        ],
    )(lp, lt, cls_targets)

    loc_loss = s[0, 0]
    num_pos = n[0, 0]
    return 0.2 * loc_loss / num_pos


# speedup vs baseline: 24.2727x; 1.7698x over previous
import functools

import jax
import jax.numpy as jnp
from jax.experimental import pallas as pl


def _ohem_body(lp_ref, lt_ref, ct_ref, s_ref, n_ref, *, ta, chunks):
    i = pl.program_id(0)
    b = i // chunks
    a0 = (i % chunks) * ta

    diff = lp_ref[0] - lt_ref[0]
    absd = jnp.abs(diff)
    sl1 = jnp.where(absd < 1.0, 0.5 * diff * diff, absd - 0.5)

    pos = (ct_ref[pl.ds(b, 1), pl.ds(a0, ta)] > 0).astype(jnp.float32)

    s_part = jnp.sum(sl1 * pos).reshape(1, 1)
    n_part = jnp.sum(pos).reshape(1, 1)

    @pl.when(i == 0)
    def _init():
        s_ref[...] = jnp.zeros((1, 1), jnp.float32)
        n_ref[...] = jnp.zeros((1, 1), jnp.float32)

    s_ref[...] += s_part
    n_ref[...] += n_part


def kernel(loc_preds, loc_targets, cls_preds, cls_targets):
    del cls_preds
    B, A, K = loc_preds.shape

    lp = jnp.transpose(loc_preds, (0, 2, 1))
    lt = jnp.transpose(loc_targets, (0, 2, 1))

    ta = 65536
    while A % ta:
        ta //= 2
    chunks = A // ta
    grid = B * chunks

    body = functools.partial(_ohem_body, ta=ta, chunks=chunks)

    s, n = pl.pallas_call(
        body,
        grid=(grid,),
        in_specs=[
            pl.BlockSpec((1, K, ta), lambda i: (i // chunks, 0, i % chunks)),
            pl.BlockSpec((1, K, ta), lambda i: (i // chunks, 0, i % chunks)),
            pl.BlockSpec((B, A), lambda i: (0, 0)),
        ],
        out_specs=[
            pl.BlockSpec((1, 1), lambda i: (0, 0)),
            pl.BlockSpec((1, 1), lambda i: (0, 0)),
        ],
        out_shape=[
            jax.ShapeDtypeStruct((1, 1), jnp.float32),
            jax.ShapeDtypeStruct((1, 1), jnp.float32),
        ],
    )(lp, lt, cls_targets)

    loc_loss = s[0, 0]
    num_pos = n[0, 0]
    return 0.2 * loc_loss / num_pos
